# baseline (speedup 1.0000x reference)
import functools

import jax
import jax.numpy as jnp
from jax import lax
from jax.experimental import pallas as pl
from jax.experimental.pallas import tpu as pltpu
from jax.experimental.pallas import tpu_sc as plsc

_B = 16384
_H = 32
_NW = 32
_BPW = _B // _NW
_NCHUNK = 4
_CHUNK = _BPW // _NCHUNK
_LMBDA = 0.0001


def _sc_body(h_hbm, t_hbm, r_hbm, ent1, ent2, rel1, rel2,
             res_out, regul_out,
             idx_h, idx_t, idx_r,
             e1h, e2h, e1t, e2t, rv1, rv2,
             dbuf, resbuf, accbuf, sem):
    nc = 2
    wid = lax.axis_index("s") * nc + lax.axis_index("c")

    pltpu.sync_copy(h_hbm.at[wid], idx_h)
    pltpu.sync_copy(t_hbm.at[wid], idx_t)
    pltpu.sync_copy(r_hbm.at[wid], idx_r)

    cps = []
    for tbl, idx, dst in ((ent1, idx_h, e1h), (ent2, idx_h, e2h),
                          (ent1, idx_t, e1t), (ent2, idx_t, e2t),
                          (rel1, idx_r, rv1), (rel2, idx_r, rv2)):
        for j in range(_NCHUNK):
            cps.append(pltpu.async_copy(
                tbl.at[idx.at[j]], dst.at[pl.ds(j * _CHUNK, _CHUNK)], sem))
    for cp in cps:
        cp.wait()

    def row_body(b, acc):
        d_off = b * 16
        tot = jnp.zeros((16,), jnp.float32)
        for half in range(2):
            sl = pl.ds(half * 16, 16)
            a = e1h[b, sl]
            bb = e2h[b, sl]
            c = e1t[b, sl]
            d = e2t[b, sl]
            p = rv1[b, sl]
            q = rv2[b, sl]
            tot = tot + (a * c + bb * d) * p + (a * d - bb * c) * q
            acc = acc + a * a + bb * bb + c * c + d * d + p * p + q * q
        dbuf[pl.ds(d_off, 16)] = tot
        return acc

    acc = lax.fori_loop(0, _BPW, row_body,
                        jnp.zeros((16,), jnp.float32), unroll=2)

    lanes16 = lax.iota(jnp.int32, 16) * 16

    def grp_body(g, carry):
        base = g * 256
        acc16 = jnp.zeros((16,), jnp.float32)
        for j in range(16):
            acc16 = acc16 + plsc.load_gather(dbuf, [base + lanes16 + j])
        resbuf[pl.ds(g * 16, 16)] = acc16
        return carry

    lax.fori_loop(0, _BPW // 16, grp_body, 0, unroll=2)

    accbuf[...] = acc
    pltpu.sync_copy(resbuf, res_out.at[wid])
    pltpu.sync_copy(accbuf, regul_out.at[wid])


@functools.partial(jax.jit, static_argnames=())
def _sc_call(h3, t3, r3, ent1, ent2, rel1, rel2):
    mesh = plsc.VectorSubcoreMesh(core_axis_name="c", subcore_axis_name="s")
    return pl.kernel(
        _sc_body,
        out_type=[
            jax.ShapeDtypeStruct((_NW, _BPW), jnp.float32),
            jax.ShapeDtypeStruct((_NW, 16), jnp.float32),
        ],
        mesh=mesh,
        compiler_params=pltpu.CompilerParams(
            needs_layout_passes=False, use_tc_tiling_on_sc=False),
        scratch_types=[
            pltpu.VMEM((_NCHUNK, _CHUNK), jnp.int32),
            pltpu.VMEM((_NCHUNK, _CHUNK), jnp.int32),
            pltpu.VMEM((_NCHUNK, _CHUNK), jnp.int32),
            pltpu.VMEM((_BPW, _H), jnp.float32),
            pltpu.VMEM((_BPW, _H), jnp.float32),
            pltpu.VMEM((_BPW, _H), jnp.float32),
            pltpu.VMEM((_BPW, _H), jnp.float32),
            pltpu.VMEM((_BPW, _H), jnp.float32),
            pltpu.VMEM((_BPW, _H), jnp.float32),
            pltpu.VMEM((_BPW * 16,), jnp.float32),
            pltpu.VMEM((_BPW,), jnp.float32),
            pltpu.VMEM((16,), jnp.float32),
            pltpu.SemaphoreType.DMA,
        ],
    )(h3, t3, r3, ent1, ent2, rel1, rel2)


def _tc_body(res_ref, y_ref, part_ref, out_ref):
    x = -(y_ref[...] * res_ref[...])
    sp = jnp.maximum(x, 0.0) + jnp.log1p(jnp.exp(-jnp.abs(x)))
    lf = jnp.sum(sp) * (1.0 / _B)
    reg = jnp.sum(part_ref[...]) * (1.0 / (_B * _H))
    out_ref[...] = jnp.reshape(lf + _LMBDA * reg, (1, 1))


def kernel(h, t, r, y, ent1, ent2, rel1, rel2):
    h3 = h.reshape(_NW, _NCHUNK, _CHUNK)
    t3 = t.reshape(_NW, _NCHUNK, _CHUNK)
    r3 = r.reshape(_NW, _NCHUNK, _CHUNK)
    res, parts = _sc_call(h3, t3, r3, ent1, ent2, rel1, rel2)
    res2 = res.reshape(128, 128)
    y2 = y.reshape(128, 128)
    out = pl.pallas_call(
        _tc_body,
        out_shape=jax.ShapeDtypeStruct((1, 1), jnp.float32),
    )(res2, y2, parts)
    return out[0, 0]
